# baseline (speedup 1.0000x reference)
import dataclasses

import jax
import jax.numpy as jnp
from jax import lax
from jax.experimental import pallas as pl
from jax.experimental.pallas import tpu as pltpu
from jax.experimental.pallas import tpu_sc as plsc

VOCAB = 100
EMBED_DIM = 16
BATCH = 16384
LANES = 16

NUM_CORES = 2
NUM_SUBCORES = 16
NUM_WORKERS = NUM_CORES * NUM_SUBCORES
B_PER_W = BATCH // NUM_WORKERS
NCHUNK = 4
C_ROWS = B_PER_W // NCHUNK
C_GROUPS = C_ROWS // LANES
GROUPS = B_PER_W // LANES


def _sc_embed(table_flat, idx):
    mesh = plsc.VectorSubcoreMesh(core_axis_name="c", subcore_axis_name="s")
    cp = pltpu.CompilerParams()
    if "needs_layout_passes" in pltpu.CompilerParams.__dataclass_fields__:
        cp = dataclasses.replace(cp, needs_layout_passes=False)

    @pl.kernel(
        mesh=mesh,
        compiler_params=cp,
        out_type=jax.ShapeDtypeStruct(
            (BATCH * EMBED_DIM // 128, 128), jnp.float32
        ),
        scratch_types=[
            pltpu.VMEM((VOCAB * EMBED_DIM,), jnp.float32),
            pltpu.VMEM((B_PER_W,), jnp.int32),
            pltpu.VMEM((B_PER_W * EMBED_DIM // 128, 128), jnp.float32),
            pltpu.SemaphoreType.DMA,
            pltpu.SemaphoreType.DMA,
            pltpu.SemaphoreType.DMA,
        ],
    )
    def k(table_hbm, idx_hbm, out_hbm, table_v, idx_v, out_v, sem_t, sem_i, sem_o):
        wid = lax.axis_index("s") * NUM_CORES + lax.axis_index("c")
        base = wid * B_PER_W
        ct = pltpu.async_copy(table_hbm, table_v, sem_t)
        ci = pltpu.async_copy(idx_hbm.at[pl.ds(base, B_PER_W)], idx_v, sem_i)
        ci.wait()

        @pl.loop(0, GROUPS)
        def _(g):
            s = pl.ds(g * LANES, LANES)
            idx_v[s] = idx_v[s] * EMBED_DIM

        ct.wait()

        iota = lax.broadcasted_iota(jnp.int32, (LANES,), 0)
        out_copies = []
        for c in range(NCHUNK):

            @pl.loop(0, 8)
            def _(s, c=c):
                for j in range(LANES):
                    r = s * 64 + c * 16 + j
                    jv = plsc.load_gather(idx_v, [jnp.broadcast_to(r, (LANES,))])
                    pos = jv + iota
                    row = plsc.load_gather(table_v, [pos])
                    out_v[c * 16 + j, pl.ds(s * EMBED_DIM, EMBED_DIM)] = row

            c_vrows = 16
            out_copies.append(
                pltpu.async_copy(
                    out_v.at[pl.ds(c * c_vrows, c_vrows), :],
                    out_hbm.at[
                        pl.ds(wid * (B_PER_W * EMBED_DIM // 128) + c * c_vrows,
                              c_vrows),
                        :,
                    ],
                    sem_o,
                )
            )
        for cc in out_copies:
            cc.wait()

    return k(table_flat, idx)


def _tc_relayout_body(x_ref, o_ref):
    for s in range(8):
        o_ref[pl.ds(s * 64, 64), :] = x_ref[:, pl.ds(s * EMBED_DIM, EMBED_DIM)]


def _tc_relayout(x2d):
    blocks = 32
    rows_in = x2d.shape[0] // blocks
    rows_out = BATCH // blocks
    return pl.pallas_call(
        _tc_relayout_body,
        grid=(blocks,),
        in_specs=[pl.BlockSpec((rows_in, 128), lambda i: (i, 0))],
        out_specs=pl.BlockSpec((rows_out, EMBED_DIM), lambda i: (i, 0)),
        out_shape=jax.ShapeDtypeStruct((BATCH, EMBED_DIM), jnp.float32),
    )(x2d)


def kernel(inputs, W):
    idx = inputs.reshape(BATCH).astype(jnp.int32)
    out2d = _sc_embed(W.reshape(VOCAB * EMBED_DIM), idx)
    return _tc_relayout(out2d)


# speedup vs baseline: 1.5607x; 1.5607x over previous
import dataclasses

import jax
import jax.numpy as jnp
from jax import lax
from jax.experimental import pallas as pl
from jax.experimental.pallas import tpu as pltpu
from jax.experimental.pallas import tpu_sc as plsc

VOCAB = 100
EMBED_DIM = 16
BATCH = 16384
LANES = 16

NUM_CORES = 2
NUM_SUBCORES = 16
NUM_WORKERS = NUM_CORES * NUM_SUBCORES
B_PER_W = BATCH // NUM_WORKERS
NCHUNK = 4
C_ROWS = B_PER_W // NCHUNK
C_GROUPS = C_ROWS // LANES
GROUPS = B_PER_W // LANES


def _sc_embed(table_flat, idx):
    mesh = plsc.VectorSubcoreMesh(core_axis_name="c", subcore_axis_name="s")
    cp = pltpu.CompilerParams()
    if "needs_layout_passes" in pltpu.CompilerParams.__dataclass_fields__:
        cp = dataclasses.replace(cp, needs_layout_passes=False)

    @pl.kernel(
        mesh=mesh,
        compiler_params=cp,
        out_type=jax.ShapeDtypeStruct((BATCH // 8, 8, EMBED_DIM), jnp.float32),
        scratch_types=[
            pltpu.VMEM((VOCAB * EMBED_DIM,), jnp.float32),
            pltpu.VMEM((B_PER_W,), jnp.int32),
            pltpu.VMEM((B_PER_W // 8, 8, EMBED_DIM), jnp.float32),
            pltpu.SemaphoreType.DMA,
            pltpu.SemaphoreType.DMA,
            pltpu.SemaphoreType.DMA,
        ],
    )
    def k(table_hbm, idx_hbm, out_hbm, table_v, idx_v, out_v, sem_t, sem_i, sem_o):
        wid = lax.axis_index("s") * NUM_CORES + lax.axis_index("c")
        base = wid * B_PER_W
        ct = pltpu.async_copy(table_hbm, table_v, sem_t)
        ci = pltpu.async_copy(idx_hbm.at[pl.ds(base, B_PER_W)], idx_v, sem_i)
        ci.wait()

        @pl.loop(0, GROUPS)
        def _(g):
            s = pl.ds(g * LANES, LANES)
            idx_v[s] = idx_v[s] * EMBED_DIM

        ct.wait()

        iota = lax.broadcasted_iota(jnp.int32, (LANES,), 0)
        out_copies = []
        for c in range(NCHUNK):

            @pl.loop(0, C_GROUPS)
            def _(g, c=c):
                for j in range(LANES):
                    r = c * C_ROWS + g * LANES + j
                    jv = plsc.load_gather(idx_v, [jnp.broadcast_to(r, (LANES,))])
                    pos = jv + iota
                    row = plsc.load_gather(table_v, [pos])
                    out_v[c * 16 + g * 2 + j // 8, j % 8, :] = row

            c_vrows = 16
            out_copies.append(
                pltpu.async_copy(
                    out_v.at[pl.ds(c * c_vrows, c_vrows)],
                    out_hbm.at[pl.ds(wid * (B_PER_W // 8) + c * c_vrows, c_vrows)],
                    sem_o,
                )
            )
        for cc in out_copies:
            cc.wait()

    return k(table_flat, idx)


def kernel(inputs, W):
    idx = inputs.reshape(BATCH).astype(jnp.int32)
    out2d = _sc_embed(W.reshape(VOCAB * EMBED_DIM), idx)
    return out2d.reshape(BATCH, EMBED_DIM)
